# baseline (speedup 1.0000x reference)
import functools

import jax
import jax.numpy as jnp
from jax import lax
from jax.experimental import pallas as pl
from jax.experimental.pallas import tpu as pltpu
from jax.experimental.pallas import tpu_sc as plsc

NUM_TOKENS = 16384
EMBED = 64
OUT = 128
LANES = 16
NC, NS = 2, 16
NW = NC * NS
TOK_PER_W = NUM_TOKENS // NW
CHUNK = 128
NCHUNK = TOK_PER_W // CHUNK
EV = EMBED // LANES

_mesh = plsc.VectorSubcoreMesh(core_axis_name="c", subcore_axis_name="s")


@functools.partial(
    pl.kernel,
    out_type=jax.ShapeDtypeStruct((NW, EMBED), jnp.float32),
    mesh=_mesh,
    scratch_types=[
        pltpu.VMEM((NCHUNK, CHUNK), jnp.int32),
        pltpu.VMEM((TOK_PER_W, EMBED), jnp.float32),
        pltpu.VMEM((EMBED,), jnp.float32),
        pltpu.SemaphoreType.DMA,
    ],
    compiler_params=pltpu.CompilerParams(use_tc_tiling_on_sc=False),
)
def _embed_partial_sums(idx_hbm, table_hbm, out_hbm, idx_v, rows_v, acc_v, sem):
    wid = lax.axis_index("s") * NC + lax.axis_index("c")
    pltpu.sync_copy(idx_hbm.at[wid], idx_v)
    copies = [
        pltpu.async_copy(
            table_hbm.at[idx_v.at[j]],
            rows_v.at[pl.ds(j * CHUNK, CHUNK)],
            sem,
        )
        for j in range(NCHUNK)
    ]
    for cp in copies:
        cp.wait()

    zero = jnp.zeros((LANES,), jnp.float32)

    def body(i, acc):
        return tuple(
            acc[c] + rows_v[i, pl.ds(c * LANES, LANES)] for c in range(EV)
        )

    acc = lax.fori_loop(0, TOK_PER_W, body, (zero,) * EV)
    for c in range(EV):
        acc_v[pl.ds(c * LANES, LANES)] = acc[c]
    pltpu.sync_copy(acc_v, out_hbm.at[wid])


def _dense_head(p_ref, w_ref, b_ref, o_ref):
    s = jnp.sum(p_ref[...], axis=0, keepdims=True) * (1.0 / NUM_TOKENS)
    o_ref[...] = (
        lax.dot_general(
            s, w_ref[...], (((1,), (1,)), ((), ())),
            preferred_element_type=jnp.float32,
        )
        + b_ref[...]
    )


def kernel(x, emb_table, fc_weight, fc_bias, extra_bias):
    idx = x.reshape(NW, NCHUNK, CHUNK)
    partials = _embed_partial_sums(idx, emb_table)
    bias = (fc_bias + extra_bias).reshape(1, OUT)
    out = pl.pallas_call(
        _dense_head,
        out_shape=jax.ShapeDtypeStruct((1, OUT), jnp.float32),
    )(partials, fc_weight, bias)
    return out


# speedup vs baseline: 3.8381x; 3.8381x over previous
import functools

import jax
import jax.numpy as jnp
from jax import lax
from jax.experimental import pallas as pl
from jax.experimental.pallas import tpu as pltpu
from jax.experimental.pallas import tpu_sc as plsc

NUM_TOKENS = 16384
VOCAB = 1_000_000
EMBED = 64
OUT = 128
LANES = 16
NC, NS = 2, 16
TOK_PER_W = NUM_TOKENS // (NC * NS)
HSIZE = 1 << 20
SLICE_PER_S = HSIZE // NS
ZB = 16384

_mesh = plsc.VectorSubcoreMesh(core_axis_name="c", subcore_axis_name="s")


@functools.partial(
    pl.kernel,
    out_type=jax.ShapeDtypeStruct((NC, HSIZE), jnp.float32),
    mesh=_mesh,
    scratch_types=[
        pltpu.VMEM((TOK_PER_W,), jnp.int32),
        pltpu.VMEM((TOK_PER_W,), jnp.float32),
        pltpu.VMEM((ZB,), jnp.float32),
        pltpu.VMEM_SHARED((HSIZE,), jnp.float32),
    ],
)
def _histogram(idx_hbm, out_hbm, idx_v, ones_v, zblk_v, hist_sh):
    cid = lax.axis_index("c")
    sid = lax.axis_index("s")

    zv = jnp.zeros((LANES,), jnp.float32)
    ov = jnp.full((LANES,), 1.0, jnp.float32)

    def fill_z(i, _):
        zblk_v[pl.ds(i * LANES, LANES)] = zv
        return 0

    lax.fori_loop(0, ZB // LANES, fill_z, 0)

    def fill_o(i, _):
        ones_v[pl.ds(i * LANES, LANES)] = ov
        return 0

    lax.fori_loop(0, TOK_PER_W // LANES, fill_o, 0)

    base = sid * SLICE_PER_S
    for r in range(SLICE_PER_S // ZB):
        pltpu.sync_copy(zblk_v, hist_sh.at[pl.ds(base + r * ZB, ZB)])
    plsc.subcore_barrier()

    pltpu.sync_copy(idx_hbm.at[cid, sid], idx_v)
    pltpu.sync_copy(ones_v, hist_sh.at[idx_v], add=True)
    plsc.subcore_barrier()

    for r in range(SLICE_PER_S // ZB):
        sl = pl.ds(base + r * ZB, ZB)
        pltpu.sync_copy(hist_sh.at[sl], out_hbm.at[cid, sl])


BK = 8192
NSTEP = (VOCAB + BK - 1) // BK


def _matvec_head(tabT_ref, c_ref, w_ref, b_ref, o_ref, acc_ref):
    k = pl.program_id(0)

    @pl.when(k == 0)
    def _():
        acc_ref[...] = jnp.zeros_like(acc_ref)

    cb = (c_ref[0, :] + c_ref[1, :]).reshape(1, BK)
    tb = tabT_ref[...]
    acc_ref[...] += lax.dot_general(
        cb, tb, (((1,), (1,)), ((), ())), preferred_element_type=jnp.float32
    )

    @pl.when(k == NSTEP - 1)
    def _():
        s = acc_ref[...] * (1.0 / NUM_TOKENS)
        o_ref[...] = (
            lax.dot_general(
                s, w_ref[...], (((1,), (1,)), ((), ())),
                preferred_element_type=jnp.float32,
            )
            + b_ref[...]
        )


def kernel(x, emb_table, fc_weight, fc_bias, extra_bias):
    idx = x.reshape(NC, NS, TOK_PER_W)
    counts = _histogram(idx)

    tabT = emb_table.T
    bias = (fc_bias + extra_bias).reshape(1, OUT)
    out = pl.pallas_call(
        _matvec_head,
        grid=(NSTEP,),
        in_specs=[
            pl.BlockSpec((EMBED, BK), lambda k: (0, k)),
            pl.BlockSpec((NC, BK), lambda k: (0, k)),
            pl.BlockSpec((OUT, EMBED), lambda k: (0, 0)),
            pl.BlockSpec((1, OUT), lambda k: (0, 0)),
        ],
        out_specs=pl.BlockSpec((1, OUT), lambda k: (0, 0)),
        out_shape=jax.ShapeDtypeStruct((1, OUT), jnp.float32),
        scratch_shapes=[pltpu.VMEM((1, EMBED), jnp.float32)],
    )(tabT, counts, fc_weight, bias)
    return out


# speedup vs baseline: 5.4956x; 1.4319x over previous
import functools

import jax
import jax.numpy as jnp
from jax import lax
from jax.experimental import pallas as pl
from jax.experimental.pallas import tpu as pltpu
from jax.experimental.pallas import tpu_sc as plsc

NUM_TOKENS = 16384
VOCAB = 1_000_000
EMBED = 64
OUT = 128
LANES = 16
NC, NS = 2, 16
TOK_PER_W = NUM_TOKENS // (NC * NS)
HSIZE = 1 << 20
SLICE_PER_S = HSIZE // NS
ZB = 16384

_mesh = plsc.VectorSubcoreMesh(core_axis_name="c", subcore_axis_name="s")


@functools.partial(
    pl.kernel,
    out_type=jax.ShapeDtypeStruct((NC, HSIZE), jnp.float32),
    mesh=_mesh,
    scratch_types=[
        pltpu.VMEM((TOK_PER_W,), jnp.int32),
        pltpu.VMEM((TOK_PER_W,), jnp.float32),
        pltpu.VMEM((ZB,), jnp.float32),
        pltpu.VMEM_SHARED((HSIZE,), jnp.float32),
    ],
)
def _histogram(idx_hbm, out_hbm, idx_v, ones_v, zblk_v, hist_sh):
    cid = lax.axis_index("c")
    sid = lax.axis_index("s")

    zv = jnp.zeros((LANES,), jnp.float32)
    ov = jnp.full((LANES,), 1.0, jnp.float32)

    def fill_z(i, _):
        zblk_v[pl.ds(i * LANES, LANES)] = zv
        return 0

    lax.fori_loop(0, ZB // LANES, fill_z, 0)

    def fill_o(i, _):
        ones_v[pl.ds(i * LANES, LANES)] = ov
        return 0

    lax.fori_loop(0, TOK_PER_W // LANES, fill_o, 0)

    base = sid * SLICE_PER_S
    for r in range(SLICE_PER_S // ZB):
        pltpu.sync_copy(zblk_v, hist_sh.at[pl.ds(base + r * ZB, ZB)])
    plsc.subcore_barrier()

    pltpu.sync_copy(idx_hbm.at[cid, sid], idx_v)
    pltpu.sync_copy(ones_v, hist_sh.at[idx_v], add=True)
    plsc.subcore_barrier()

    for r in range(SLICE_PER_S // ZB):
        sl = pl.ds(base + r * ZB, ZB)
        pltpu.sync_copy(hist_sh.at[sl], out_hbm.at[cid, sl])


BK = 32768
NSTEP = (VOCAB + BK - 1) // BK


def _matvec_head(tabT_ref, c_ref, w_ref, b_ref, o_ref, acc_ref):
    k = pl.program_id(0)

    @pl.when(k == 0)
    def _():
        acc_ref[...] = jnp.zeros_like(acc_ref)

    cb = (c_ref[0, :] + c_ref[1, :]).reshape(1, BK)
    acc_ref[...] += tabT_ref[...] * cb

    @pl.when(k == NSTEP - 1)
    def _():
        s = jnp.sum(acc_ref[...], axis=1).reshape(1, EMBED) * (1.0 / NUM_TOKENS)
        o_ref[...] = (
            lax.dot_general(
                s, w_ref[...], (((1,), (1,)), ((), ())),
                preferred_element_type=jnp.float32,
            )
            + b_ref[...]
        )


def kernel(x, emb_table, fc_weight, fc_bias, extra_bias):
    idx = x.reshape(NC, NS, TOK_PER_W)
    counts = _histogram(idx)

    tabT = emb_table.T
    bias = (fc_bias + extra_bias).reshape(1, OUT)
    out = pl.pallas_call(
        _matvec_head,
        grid=(NSTEP,),
        in_specs=[
            pl.BlockSpec((EMBED, BK), lambda k: (0, k)),
            pl.BlockSpec((NC, BK), lambda k: (0, k)),
            pl.BlockSpec((OUT, EMBED), lambda k: (0, 0)),
            pl.BlockSpec((1, OUT), lambda k: (0, 0)),
        ],
        out_specs=pl.BlockSpec((1, OUT), lambda k: (0, 0)),
        out_shape=jax.ShapeDtypeStruct((1, OUT), jnp.float32),
        scratch_shapes=[pltpu.VMEM((EMBED, BK), jnp.float32)],
    )(tabT, counts, fc_weight, bias)
    return out
